# baseline (speedup 1.0000x reference)
import functools

import jax
import jax.numpy as jnp
from jax import lax
from jax.experimental import pallas as pl
from jax.experimental.pallas import tpu as pltpu
from jax.experimental.pallas import tpu_sc as plsc

N = 10000
D = 128
E = 320000
NPAD = 10240
NC = 2
NS = 16
NW = NC * NS
G = 128
K = 80
KC = 16
NCH = K // KC
EPAD = NW * K * G
RPS = NPAD // NS
DEG_W = 16
MB = 1024


def _sc_mesh():
    return plsc.VectorSubcoreMesh(core_axis_name="c", subcore_axis_name="s")


def _deg_sc(dst2d, zeros_init, ones_rows):
    @functools.partial(
        pl.kernel,
        out_type=jax.ShapeDtypeStruct((NC, NPAD, D), jnp.float32),
        mesh=_sc_mesh(),
        scratch_types=[
            pltpu.VMEM((K, G), jnp.int32),
            pltpu.VMEM((G, D), jnp.float32),
            pltpu.VMEM_SHARED((NPAD, D), jnp.float32),
        ],
    )
    def k(dst_hbm, zero_hbm, ones_hbm, out_hbm, dst_v, ones_v, acc):
        c = lax.axis_index("c")
        s = lax.axis_index("s")
        wid = s * NC + c
        pltpu.sync_copy(zero_hbm.at[pl.ds(s * RPS, RPS)],
                        acc.at[pl.ds(s * RPS, RPS)])
        pltpu.sync_copy(ones_hbm, ones_v)
        pltpu.sync_copy(dst_hbm.at[pl.ds(wid * K, K)], dst_v)
        plsc.subcore_barrier()

        def body(j, carry):
            pltpu.sync_copy(ones_v, acc.at[dst_v.at[j]], add=True)
            return carry

        lax.fori_loop(0, K, body, 0)
        plsc.subcore_barrier()
        pltpu.sync_copy(acc.at[pl.ds(s * RPS, RPS)],
                        out_hbm.at[c, pl.ds(s * RPS, RPS)])

    return k(dst2d, zeros_init, ones_rows)


def _segsum_sc(y, src2d, dst2d):
    @functools.partial(
        pl.kernel,
        out_type=jax.ShapeDtypeStruct((NC, NPAD, D), jnp.float32),
        mesh=_sc_mesh(),
        scratch_types=[
            pltpu.VMEM((KC, G), jnp.int32),
            pltpu.VMEM((KC, G), jnp.int32),
            pltpu.VMEM((G, D), jnp.float32),
            pltpu.VMEM((G, D), jnp.float32),
            pltpu.VMEM_SHARED((NPAD, D), jnp.float32),
            pltpu.SemaphoreType.DMA,
            pltpu.SemaphoreType.DMA,
        ],
    )
    def k(y_hbm, src_hbm, dst_hbm, out_hbm, src_v, dst_v, rows0, rows1,
          acc, sem0, sem1):
        c = lax.axis_index("c")
        s = lax.axis_index("s")
        wid = s * NC + c
        bufs = (rows0, rows1)
        sems = (sem0, sem1)
        pltpu.sync_copy(y_hbm.at[pl.ds(s * RPS, RPS)],
                        acc.at[pl.ds(s * RPS, RPS)])
        plsc.subcore_barrier()

        def fire(jj, b):
            for h in range(2):
                pltpu.make_async_copy(
                    y_hbm.at[src_v.at[jj, pl.ds(h * 64, 64)]],
                    bufs[b].at[pl.ds(h * 64, 64)], sems[b]).start()

        def chunk(ch, carry):
            base = wid * K + ch * KC
            pltpu.sync_copy(src_hbm.at[pl.ds(base, KC)], src_v)
            pltpu.sync_copy(dst_hbm.at[pl.ds(base, KC)], dst_v)
            for b in range(2):
                fire(b, b)

            def body(i, c2):
                j = i * 2
                for b in range(2):
                    jj = j + b
                    pltpu.make_async_copy(
                        y_hbm.at[src_v.at[jj]], bufs[b], sems[b]).wait()
                    pltpu.sync_copy(bufs[b], acc.at[dst_v.at[jj]], add=True)

                    @pl.when(jj + 2 < KC)
                    def _():
                        fire(jj + 2, b)
                return c2

            lax.fori_loop(0, KC // 2, body, 0)
            return carry

        lax.fori_loop(0, NCH, chunk, 0)
        plsc.subcore_barrier()
        pltpu.sync_copy(acc.at[pl.ds(s * RPS, RPS)],
                        out_hbm.at[c, pl.ds(s * RPS, RPS)])

    return k(y, src2d, dst2d)


def _tc1_body(deg_ref, x_ref, w_ref, y_ref, dinv_ref):
    i = pl.program_id(0)
    degsum = deg_ref[0] + deg_ref[1]
    deg = degsum[:, 0:1] + 1.0
    row = i * MB + lax.broadcasted_iota(jnp.int32, (MB, 1), 0)
    dinv = jnp.where(row < N, lax.rsqrt(deg), 0.0)
    xw = jnp.dot(x_ref[...], w_ref[...], preferred_element_type=jnp.float32)
    y_ref[...] = xw * dinv
    dinv_ref[...] = jnp.broadcast_to(dinv, (MB, D))


def _tc1(deg_parts, x_pad, W1):
    grid = NPAD // MB
    return pl.pallas_call(
        _tc1_body,
        grid=(grid,),
        in_specs=[
            pl.BlockSpec((NC, MB, D), lambda i: (0, i, 0)),
            pl.BlockSpec((MB, D), lambda i: (i, 0)),
            pl.BlockSpec((D, D), lambda i: (0, 0)),
        ],
        out_specs=[
            pl.BlockSpec((MB, D), lambda i: (i, 0)),
            pl.BlockSpec((MB, D), lambda i: (i, 0)),
        ],
        out_shape=[
            jax.ShapeDtypeStruct((NPAD, D), jnp.float32),
            jax.ShapeDtypeStruct((NPAD, D), jnp.float32),
        ],
    )(deg_parts, x_pad, W1)


def _tc2_body(p_ref, y_ref, dinv_ref, b_ref, w_ref, y2_ref):
    i = pl.program_id(0)
    ssum = p_ref[0] + p_ref[1] - y_ref[...]
    h = jnp.maximum(dinv_ref[...] * ssum + b_ref[...], 0.0)
    row = i * MB + lax.broadcasted_iota(jnp.int32, (MB, 1), 0)
    h = jnp.where(row < N, h, 0.0)
    y2_ref[...] = jnp.dot(h, w_ref[...],
                          preferred_element_type=jnp.float32) * dinv_ref[...]


def _tc2(parts, y1, dinv_bc, b1, W2):
    grid = NPAD // MB
    return pl.pallas_call(
        _tc2_body,
        grid=(grid,),
        in_specs=[
            pl.BlockSpec((NC, MB, D), lambda i: (0, i, 0)),
            pl.BlockSpec((MB, D), lambda i: (i, 0)),
            pl.BlockSpec((MB, D), lambda i: (i, 0)),
            pl.BlockSpec((1, D), lambda i: (0, 0)),
            pl.BlockSpec((D, D), lambda i: (0, 0)),
        ],
        out_specs=pl.BlockSpec((MB, D), lambda i: (i, 0)),
        out_shape=jax.ShapeDtypeStruct((NPAD, D), jnp.float32),
    )(parts, y1, dinv_bc, b1, W2)


def _tc3_body(p_ref, y_ref, dinv_ref, b_ref, o_ref):
    ssum = p_ref[0] + p_ref[1] - y_ref[...]
    o_ref[...] = jnp.maximum(dinv_ref[...] * ssum + b_ref[...], 0.0)


def _tc3(parts, y2, dinv_bc, b2):
    blk = 1000
    grid = N // blk
    return pl.pallas_call(
        _tc3_body,
        grid=(grid,),
        in_specs=[
            pl.BlockSpec((NC, blk, D), lambda i: (0, i, 0)),
            pl.BlockSpec((blk, D), lambda i: (i, 0)),
            pl.BlockSpec((blk, D), lambda i: (i, 0)),
            pl.BlockSpec((1, D), lambda i: (0, 0)),
        ],
        out_specs=pl.BlockSpec((blk, D), lambda i: (i, 0)),
        out_shape=jax.ShapeDtypeStruct((N, D), jnp.float32),
    )(parts, y2, dinv_bc, b2)


def kernel(x, edge_index, W1, b1, W2, b2):
    src = edge_index[0].astype(jnp.int32)
    dst = edge_index[1].astype(jnp.int32)
    npad_e = EPAD - E
    srcp = jnp.concatenate(
        [src, jnp.full((npad_e,), N, jnp.int32)]).reshape(NW * K, G)
    dstp = jnp.concatenate(
        [dst, jnp.full((npad_e,), N, jnp.int32)]).reshape(NW * K, G)
    x_pad = jnp.pad(x, ((0, NPAD - N), (0, 0)))

    zeros_init = jnp.zeros((NPAD, D), jnp.float32)
    ones_rows = jnp.ones((G, D), jnp.float32)

    deg_parts = _deg_sc(dstp, zeros_init, ones_rows)
    y1, dinv_bc = _tc1(deg_parts, x_pad, W1)
    s1 = _segsum_sc(y1, srcp, dstp)
    y2 = _tc2(s1, y1, dinv_bc, b1.reshape(1, D), W2)
    s2 = _segsum_sc(y2, srcp, dstp)
    return _tc3(s2, y2, dinv_bc, b2.reshape(1, D))


# speedup vs baseline: 1.2419x; 1.2419x over previous
import functools

import jax
import jax.numpy as jnp
from jax import lax
from jax.experimental import pallas as pl
from jax.experimental.pallas import tpu as pltpu
from jax.experimental.pallas import tpu_sc as plsc

N = 10000
D = 128
E = 320000
NY = 10048
MB = 1256
HALF = 5120
ACC = 5128
NDEG = 10240
EPAD = 327680
EROWS = EPAD // 128
ROWS = EROWS // 16
KC = 2
NCH = ROWS // KC
K = 80


def _sc_mesh():
    return plsc.VectorSubcoreMesh(core_axis_name="c", subcore_axis_name="s")


def _deg_sc(dst2d, zeros_init, ones_rows):
    @functools.partial(
        pl.kernel,
        out_type=jax.ShapeDtypeStruct((2, NDEG, D), jnp.float32),
        mesh=_sc_mesh(),
        scratch_types=[
            pltpu.VMEM((K, 128), jnp.int32),
            pltpu.VMEM((128, D), jnp.float32),
            pltpu.VMEM_SHARED((NDEG, D), jnp.float32),
        ],
    )
    def k(dst_hbm, zero_hbm, ones_hbm, out_hbm, dst_v, ones_v, acc):
        c = lax.axis_index("c")
        s = lax.axis_index("s")
        wid = s * 2 + c
        rps = NDEG // 16
        pltpu.sync_copy(zero_hbm.at[pl.ds(s * rps, rps)],
                        acc.at[pl.ds(s * rps, rps)])
        pltpu.sync_copy(ones_hbm, ones_v)
        pltpu.sync_copy(dst_hbm.at[pl.ds(wid * K, K)], dst_v)
        plsc.subcore_barrier()

        def body(j, carry):
            pltpu.sync_copy(ones_v, acc.at[dst_v.at[j]], add=True)
            return carry

        lax.fori_loop(0, K, body, 0)
        plsc.subcore_barrier()
        pltpu.sync_copy(acc.at[pl.ds(s * rps, rps)],
                        out_hbm.at[c, pl.ds(s * rps, rps)])

    return k(dst2d, zeros_init, ones_rows)


def _segsum_sc(y, src2d, dst3):
    @functools.partial(
        pl.kernel,
        out_type=jax.ShapeDtypeStruct((2, HALF, D), jnp.float32),
        mesh=_sc_mesh(),
        scratch_types=[
            pltpu.VMEM((KC, 128), jnp.int32),
            pltpu.VMEM((KC, 128), jnp.int32),
            pltpu.VMEM((32, D), jnp.float32),
            pltpu.VMEM((32, D), jnp.float32),
            pltpu.VMEM_SHARED((NY, D), jnp.float32),
            pltpu.VMEM_SHARED((ACC, D), jnp.float32),
            pltpu.SemaphoreType.DMA,
            pltpu.SemaphoreType.DMA,
        ],
    )
    def k(y_hbm, src_hbm, dst3_hbm, out_hbm, src_v, dst_v, buf0, buf1,
          y_sp, acc, sem0, sem1):
        c = lax.axis_index("c")
        s = lax.axis_index("s")
        pltpu.sync_copy(y_hbm.at[pl.ds(s * 624, 624)],
                        y_sp.at[pl.ds(s * 624, 624)])

        @pl.when(s < 8)
        def _():
            pltpu.sync_copy(y_hbm.at[pl.ds(9984 + s * 8, 8)],
                            y_sp.at[pl.ds(9984 + s * 8, 8)])

        pltpu.sync_copy(y_hbm.at[pl.ds(c * HALF + s * 304, 304)],
                        acc.at[pl.ds(s * 304, 304)])

        @pl.when(c == 0)
        def _():
            pltpu.sync_copy(y_hbm.at[pl.ds(4864 + s * 16, 16)],
                            acc.at[pl.ds(4864 + s * 16, 16)])

        @pl.when((c == 1) & (s < 8))
        def _():
            pltpu.sync_copy(y_hbm.at[pl.ds(HALF + 4864 + s * 8, 8)],
                            acc.at[pl.ds(4864 + s * 8, 8)])

        plsc.subcore_barrier()

        bufs = (buf0, buf1)
        sems = (sem0, sem1)

        def fire(j, h, b):
            pltpu.make_async_copy(
                y_sp.at[src_v.at[j, pl.ds(h * 32, 32)]],
                bufs[b], sems[b]).start()

        def chunk(ch, carry):
            base = s * ROWS + ch * KC
            pltpu.sync_copy(src_hbm.at[pl.ds(base, KC)], src_v)
            pltpu.sync_copy(dst3_hbm.at[c, pl.ds(base, KC)], dst_v)
            fire(0, 0, 0)
            fire(0, 1, 1)
            for g in range(4 * KC):
                j = g // 4
                h = g % 4
                b = g % 2
                pltpu.make_async_copy(
                    y_sp.at[src_v.at[j, pl.ds(h * 32, 32)]],
                    bufs[b], sems[b]).wait()
                pltpu.sync_copy(bufs[b],
                                acc.at[dst_v.at[j, pl.ds(h * 32, 32)]],
                                add=True)
                if g + 2 < 4 * KC:
                    g2 = g + 2
                    fire(g2 // 4, g2 % 4, b)
            return carry

        lax.fori_loop(0, NCH, chunk, 0)
        plsc.subcore_barrier()
        pltpu.sync_copy(acc.at[pl.ds(s * 320, 320)],
                        out_hbm.at[c, pl.ds(s * 320, 320)])

    return k(y, src2d, dst3)


def _tc1_body(deg_ref, x_ref, w_ref, d_ref, y_ref, dinv_ref, da_ref, db_ref):
    i = pl.program_id(0)
    degsum = deg_ref[0] + deg_ref[1]
    deg = degsum[:, 0:1] + 1.0
    row = i * MB + lax.broadcasted_iota(jnp.int32, (MB, 1), 0)
    dinv = jnp.where(row < N, lax.rsqrt(deg), 0.0)
    xw = jnp.dot(x_ref[...], w_ref[...], preferred_element_type=jnp.float32)
    y_ref[...] = xw * dinv
    dinv_ref[...] = jnp.broadcast_to(dinv, (MB, D))
    dst = d_ref[...]
    eb = EROWS // 8
    junk = HALF + (lax.broadcasted_iota(jnp.int32, (eb, 128), 1) % 8)
    da_ref[...] = jnp.where(dst < HALF, dst, junk)
    db_ref[...] = jnp.where(dst >= HALF, dst - HALF, junk)


def _tc1(deg_parts, x_pad, W1, dst2d):
    eb = EROWS // 8
    return pl.pallas_call(
        _tc1_body,
        grid=(NY // MB,),
        in_specs=[
            pl.BlockSpec((2, MB, D), lambda i: (0, i, 0)),
            pl.BlockSpec((MB, D), lambda i: (i, 0)),
            pl.BlockSpec((D, D), lambda i: (0, 0)),
            pl.BlockSpec((eb, 128), lambda i: (i, 0)),
        ],
        out_specs=[
            pl.BlockSpec((MB, D), lambda i: (i, 0)),
            pl.BlockSpec((MB, D), lambda i: (i, 0)),
            pl.BlockSpec((eb, 128), lambda i: (i, 0)),
            pl.BlockSpec((eb, 128), lambda i: (i, 0)),
        ],
        out_shape=[
            jax.ShapeDtypeStruct((NY, D), jnp.float32),
            jax.ShapeDtypeStruct((NY, D), jnp.float32),
            jax.ShapeDtypeStruct((EROWS, 128), jnp.int32),
            jax.ShapeDtypeStruct((EROWS, 128), jnp.int32),
        ],
    )(deg_parts, x_pad, W1, dst2d)


def _tc2_body(s_ref, dinv_ref, b_ref, w_ref, y2_ref):
    i = pl.program_id(0)
    h = jnp.maximum(dinv_ref[...] * s_ref[...] + b_ref[...], 0.0)
    row = i * MB + lax.broadcasted_iota(jnp.int32, (MB, 1), 0)
    h = jnp.where(row < N, h, 0.0)
    y2_ref[...] = jnp.dot(h, w_ref[...],
                          preferred_element_type=jnp.float32) * dinv_ref[...]


def _tc2(s_full, dinv_bc, b1, W2):
    return pl.pallas_call(
        _tc2_body,
        grid=(NY // MB,),
        in_specs=[
            pl.BlockSpec((MB, D), lambda i: (i, 0)),
            pl.BlockSpec((MB, D), lambda i: (i, 0)),
            pl.BlockSpec((1, D), lambda i: (0, 0)),
            pl.BlockSpec((D, D), lambda i: (0, 0)),
        ],
        out_specs=pl.BlockSpec((MB, D), lambda i: (i, 0)),
        out_shape=jax.ShapeDtypeStruct((NY, D), jnp.float32),
    )(s_full, dinv_bc, b1, W2)


def _tc3_body(s_ref, dinv_ref, b_ref, o_ref):
    o_ref[...] = jnp.maximum(dinv_ref[...] * s_ref[...] + b_ref[...], 0.0)


def _tc3(s_full, dinv_bc, b2):
    blk = 1000
    return pl.pallas_call(
        _tc3_body,
        grid=(N // blk,),
        in_specs=[
            pl.BlockSpec((blk, D), lambda i: (i, 0)),
            pl.BlockSpec((blk, D), lambda i: (i, 0)),
            pl.BlockSpec((1, D), lambda i: (0, 0)),
        ],
        out_specs=pl.BlockSpec((blk, D), lambda i: (i, 0)),
        out_shape=jax.ShapeDtypeStruct((N, D), jnp.float32),
    )(s_full, dinv_bc, b2)


def kernel(x, edge_index, W1, b1, W2, b2):
    src = edge_index[0].astype(jnp.int32)
    dst = edge_index[1].astype(jnp.int32)
    npad_e = EPAD - E
    ip = jnp.arange(npad_e, dtype=jnp.int32)
    srcp = jnp.concatenate([src, N + (ip % 48)]).reshape(EROWS, 128)
    dstp = jnp.concatenate([dst, N + (ip % 240)]).reshape(EROWS, 128)
    x_pad = jnp.pad(x, ((0, NY - N), (0, 0)))

    zeros_init = jnp.zeros((NDEG, D), jnp.float32)
    ones_rows = jnp.ones((128, D), jnp.float32)

    deg_parts = _deg_sc(dstp, zeros_init, ones_rows)
    y1, dinv_bc, dstA, dstB = _tc1(deg_parts, x_pad, W1, dstp)
    dst3 = jnp.stack([dstA, dstB])
    s1 = _segsum_sc(y1, srcp, dst3).reshape(2 * HALF, D)
    y2 = _tc2(s1, dinv_bc, b1.reshape(1, D), W2)
    s2 = _segsum_sc(y2, srcp, dst3).reshape(2 * HALF, D)
    return _tc3(s2, dinv_bc, b2.reshape(1, D))


# speedup vs baseline: 1.3794x; 1.1108x over previous
import functools

import jax
import jax.numpy as jnp
from jax import lax
from jax.experimental import pallas as pl
from jax.experimental.pallas import tpu as pltpu
from jax.experimental.pallas import tpu_sc as plsc

N = 10000
D = 128
E = 320000
NY = 10048
MB = 1256
HALF = 5120
ACC = 5152
NDEG = 10240
EPAD = 327680
EROWS = EPAD // 128
ROWS = EROWS // 16
KC = 2
NCH = ROWS // KC
K = 80


def _sc_mesh():
    return plsc.VectorSubcoreMesh(core_axis_name="c", subcore_axis_name="s")


def _deg_sc(dst2d, zeros_init, ones_rows):
    @functools.partial(
        pl.kernel,
        out_type=jax.ShapeDtypeStruct((2, NDEG, D), jnp.float32),
        mesh=_sc_mesh(),
        scratch_types=[
            pltpu.VMEM((K, 128), jnp.int32),
            pltpu.VMEM((128, D), jnp.float32),
            pltpu.VMEM_SHARED((NDEG, D), jnp.float32),
        ],
    )
    def k(dst_hbm, zero_hbm, ones_hbm, out_hbm, dst_v, ones_v, acc):
        c = lax.axis_index("c")
        s = lax.axis_index("s")
        wid = s * 2 + c
        rps = NDEG // 16
        pltpu.sync_copy(zero_hbm.at[pl.ds(s * rps, rps)],
                        acc.at[pl.ds(s * rps, rps)])
        pltpu.sync_copy(ones_hbm, ones_v)
        pltpu.sync_copy(dst_hbm.at[pl.ds(wid * K, K)], dst_v)
        plsc.subcore_barrier()

        def body(j, carry):
            pltpu.sync_copy(ones_v, acc.at[dst_v.at[j]], add=True)
            return carry

        lax.fori_loop(0, K, body, 0)
        plsc.subcore_barrier()
        pltpu.sync_copy(acc.at[pl.ds(s * rps, rps)],
                        out_hbm.at[c, pl.ds(s * rps, rps)])

    return k(dst2d, zeros_init, ones_rows)


def _segsum_sc(y, src2d, dst3):
    @functools.partial(
        pl.kernel,
        out_type=jax.ShapeDtypeStruct((2, HALF, D), jnp.float32),
        mesh=_sc_mesh(),
        scratch_types=[
            pltpu.VMEM((KC, 128), jnp.int32),
            pltpu.VMEM((KC, 128), jnp.int32),
            pltpu.VMEM((KC, 128), jnp.int32),
            pltpu.VMEM((KC, 128), jnp.int32),
            pltpu.VMEM((32, D), jnp.float32),
            pltpu.VMEM((32, D), jnp.float32),
            pltpu.VMEM_SHARED((NY, D), jnp.float32),
            pltpu.VMEM_SHARED((ACC, D), jnp.float32),
            pltpu.SemaphoreType.DMA,
            pltpu.SemaphoreType.DMA,
            pltpu.SemaphoreType.DMA,
            pltpu.SemaphoreType.DMA,
        ],
    )
    def k(y_hbm, src_hbm, dst3_hbm, out_hbm, src_v0, dst_v0, src_v1, dst_v1,
          buf0, buf1, y_sp, acc, sem0, sem1, isem0, isem1):
        c = lax.axis_index("c")
        s = lax.axis_index("s")
        pltpu.sync_copy(y_hbm.at[pl.ds(s * 624, 624)],
                        y_sp.at[pl.ds(s * 624, 624)])

        @pl.when(s < 8)
        def _():
            pltpu.sync_copy(y_hbm.at[pl.ds(9984 + s * 8, 8)],
                            y_sp.at[pl.ds(9984 + s * 8, 8)])

        pltpu.sync_copy(y_hbm.at[pl.ds(c * HALF + s * 304, 304)],
                        acc.at[pl.ds(s * 304, 304)])

        @pl.when(c == 0)
        def _():
            pltpu.sync_copy(y_hbm.at[pl.ds(4864 + s * 16, 16)],
                            acc.at[pl.ds(4864 + s * 16, 16)])

        @pl.when((c == 1) & (s < 8))
        def _():
            pltpu.sync_copy(y_hbm.at[pl.ds(HALF + 4864 + s * 8, 8)],
                            acc.at[pl.ds(4864 + s * 8, 8)])

        plsc.subcore_barrier()

        bufs = (buf0, buf1)
        sems = (sem0, sem1)
        idx = ((src_v0, dst_v0, isem0), (src_v1, dst_v1, isem1))

        def idx_start(ch, p):
            sv, dv, isem = idx[p]
            base = s * ROWS + ch * KC
            pltpu.make_async_copy(src_hbm.at[pl.ds(base, KC)], sv, isem).start()
            pltpu.make_async_copy(dst3_hbm.at[c, pl.ds(base, KC)], dv,
                                  isem).start()

        def idx_wait(ch, p):
            sv, dv, isem = idx[p]
            base = s * ROWS + ch * KC
            pltpu.make_async_copy(src_hbm.at[pl.ds(base, KC)], sv, isem).wait()
            pltpu.make_async_copy(dst3_hbm.at[c, pl.ds(base, KC)], dv,
                                  isem).wait()

        def fire(sv, j, h, b):
            pltpu.make_async_copy(
                y_sp.at[sv.at[j, pl.ds(h * 32, 32)]],
                bufs[b], sems[b]).start()

        idx_start(0, 0)
        idx_start(1, 1)

        def chunk_body(ch, p):
            sv, dv, _ = idx[p]
            idx_wait(ch, p)
            fire(sv, 0, 0, 0)
            fire(sv, 0, 1, 1)
            for g in range(4 * KC):
                j = g // 4
                h = g % 4
                b = g % 2
                pltpu.make_async_copy(
                    y_sp.at[sv.at[j, pl.ds(h * 32, 32)]],
                    bufs[b], sems[b]).wait()
                pltpu.sync_copy(bufs[b],
                                acc.at[dv.at[j, pl.ds(h * 32, 32)]],
                                add=True)
                if g + 2 < 4 * KC:
                    g2 = g + 2
                    fire(sv, g2 // 4, g2 % 4, b)

            @pl.when(ch + 2 < NCH)
            def _():
                idx_start(ch + 2, p)

        def pair(i, carry):
            chunk_body(i * 2, 0)
            chunk_body(i * 2 + 1, 1)
            return carry

        lax.fori_loop(0, NCH // 2, pair, 0)
        plsc.subcore_barrier()
        pltpu.sync_copy(acc.at[pl.ds(s * 320, 320)],
                        out_hbm.at[c, pl.ds(s * 320, 320)])

    return k(y, src2d, dst3)


def _tc1_body(deg_ref, x_ref, w_ref, d_ref, y_ref, dinv_ref, da_ref, db_ref):
    i = pl.program_id(0)
    degsum = deg_ref[0] + deg_ref[1]
    deg = degsum[:, 0:1] + 1.0
    row = i * MB + lax.broadcasted_iota(jnp.int32, (MB, 1), 0)
    dinv = jnp.where(row < N, lax.rsqrt(deg), 0.0)
    xw = jnp.dot(x_ref[...], w_ref[...], preferred_element_type=jnp.float32)
    y_ref[...] = xw * dinv
    dinv_ref[...] = jnp.broadcast_to(dinv, (MB, D))
    dst = d_ref[...]
    eb = EROWS // 8
    junk = HALF + (lax.broadcasted_iota(jnp.int32, (eb, 128), 1) % 32)
    da_ref[...] = jnp.where(dst < HALF, dst, junk)
    db_ref[...] = jnp.where(dst >= HALF, dst - HALF, junk)


def _tc1(deg_parts, x_pad, W1, dst2d):
    eb = EROWS // 8
    return pl.pallas_call(
        _tc1_body,
        grid=(NY // MB,),
        in_specs=[
            pl.BlockSpec((2, MB, D), lambda i: (0, i, 0)),
            pl.BlockSpec((MB, D), lambda i: (i, 0)),
            pl.BlockSpec((D, D), lambda i: (0, 0)),
            pl.BlockSpec((eb, 128), lambda i: (i, 0)),
        ],
        out_specs=[
            pl.BlockSpec((MB, D), lambda i: (i, 0)),
            pl.BlockSpec((MB, D), lambda i: (i, 0)),
            pl.BlockSpec((eb, 128), lambda i: (i, 0)),
            pl.BlockSpec((eb, 128), lambda i: (i, 0)),
        ],
        out_shape=[
            jax.ShapeDtypeStruct((NY, D), jnp.float32),
            jax.ShapeDtypeStruct((NY, D), jnp.float32),
            jax.ShapeDtypeStruct((EROWS, 128), jnp.int32),
            jax.ShapeDtypeStruct((EROWS, 128), jnp.int32),
        ],
    )(deg_parts, x_pad, W1, dst2d)


def _tc2_body(s_ref, dinv_ref, b_ref, w_ref, y2_ref):
    i = pl.program_id(0)
    h = jnp.maximum(dinv_ref[...] * s_ref[...] + b_ref[...], 0.0)
    row = i * MB + lax.broadcasted_iota(jnp.int32, (MB, 1), 0)
    h = jnp.where(row < N, h, 0.0)
    y2_ref[...] = jnp.dot(h, w_ref[...],
                          preferred_element_type=jnp.float32) * dinv_ref[...]


def _tc2(s_full, dinv_bc, b1, W2):
    return pl.pallas_call(
        _tc2_body,
        grid=(NY // MB,),
        in_specs=[
            pl.BlockSpec((MB, D), lambda i: (i, 0)),
            pl.BlockSpec((MB, D), lambda i: (i, 0)),
            pl.BlockSpec((1, D), lambda i: (0, 0)),
            pl.BlockSpec((D, D), lambda i: (0, 0)),
        ],
        out_specs=pl.BlockSpec((MB, D), lambda i: (i, 0)),
        out_shape=jax.ShapeDtypeStruct((NY, D), jnp.float32),
    )(s_full, dinv_bc, b1, W2)


def _tc3_body(s_ref, dinv_ref, b_ref, o_ref):
    o_ref[...] = jnp.maximum(dinv_ref[...] * s_ref[...] + b_ref[...], 0.0)


def _tc3(s_full, dinv_bc, b2):
    blk = 1000
    return pl.pallas_call(
        _tc3_body,
        grid=(N // blk,),
        in_specs=[
            pl.BlockSpec((blk, D), lambda i: (i, 0)),
            pl.BlockSpec((blk, D), lambda i: (i, 0)),
            pl.BlockSpec((1, D), lambda i: (0, 0)),
        ],
        out_specs=pl.BlockSpec((blk, D), lambda i: (i, 0)),
        out_shape=jax.ShapeDtypeStruct((N, D), jnp.float32),
    )(s_full, dinv_bc, b2)


def kernel(x, edge_index, W1, b1, W2, b2):
    src = edge_index[0].astype(jnp.int32)
    dst = edge_index[1].astype(jnp.int32)
    npad_e = EPAD - E
    ip = jnp.arange(npad_e, dtype=jnp.int32)
    srcp = jnp.concatenate([src, N + (ip % 48)]).reshape(EROWS, 128)
    dstp = jnp.concatenate([dst, N + (ip % 240)]).reshape(EROWS, 128)
    x_pad = jnp.pad(x, ((0, NY - N), (0, 0)))

    zeros_init = jnp.zeros((NDEG, D), jnp.float32)
    ones_rows = jnp.ones((128, D), jnp.float32)

    deg_parts = _deg_sc(dstp, zeros_init, ones_rows)
    y1, dinv_bc, dstA, dstB = _tc1(deg_parts, x_pad, W1, dstp)
    dst3 = jnp.stack([dstA, dstB])
    s1 = _segsum_sc(y1, srcp, dst3).reshape(2 * HALF, D)
    y2 = _tc2(s1, dinv_bc, b1.reshape(1, D), W2)
    s2 = _segsum_sc(y2, srcp, dst3).reshape(2 * HALF, D)
    return _tc3(s2, dinv_bc, b2.reshape(1, D))
